# Initial kernel scaffold; baseline (speedup 1.0000x reference)
import jax
import jax.numpy as jnp
from jax.experimental import pallas as pl


def kernel(features, laplacianMat_indices, laplacianMat_values, selfLoop):
    raise NotImplementedError("write your pallas kernel here")


# speedup vs baseline: 4.4745x; 4.4745x over previous
import functools

import jax
import jax.numpy as jnp
from jax import lax
from jax.experimental import pallas as pl
from jax.experimental.pallas import tpu as pltpu
from jax.experimental.pallas import tpu_sc as plsc

N = 10000
E = 320000
D = 128
L = 16
NC = 2
NS = 16
NW = NC * NS
E_PER_W = E // NW
K = 80
CHUNKS = E_PER_W // K
NPAD = 10240
ROWS_PER_TILE = NPAD // NS
ZROWS = 128


def _sc_partials(features, rows, cols, values):
    mesh = plsc.VectorSubcoreMesh(
        core_axis_name="c", subcore_axis_name="s", num_cores=NC, num_subcores=NS
    )

    @functools.partial(
        pl.kernel,
        out_type=jax.ShapeDtypeStruct((NC, N, D), jnp.float32),
        mesh=mesh,
        scratch_types=[
            pltpu.VMEM((K,), jnp.int32),
            pltpu.VMEM((K,), jnp.int32),
            pltpu.VMEM((K,), jnp.float32),
            pltpu.VMEM((K, D), jnp.float32),
            pltpu.VMEM((ZROWS, D), jnp.float32),
            pltpu.VMEM_SHARED((NPAD, D), jnp.float32),
            pltpu.SemaphoreType.DMA,
        ],
    )
    def k(feat_hbm, row_hbm, col_hbm, val_hbm, out_hbm,
          cols_v, rows_v, vals_v, gath_v, zbuf_v, acc_sh, sem):
        cid = lax.axis_index("c")
        sid = lax.axis_index("s")

        def zrow(i, carry):
            for j in range(D // L):
                zbuf_v[i, pl.ds(j * L, L)] = jnp.zeros((L,), jnp.float32)
            return carry
        lax.fori_loop(0, ZROWS, zrow, 0)
        base_row = sid * ROWS_PER_TILE
        for z in range(ROWS_PER_TILE // ZROWS):
            pltpu.sync_copy(zbuf_v, acc_sh.at[pl.ds(base_row + z * ZROWS, ZROWS)])
        plsc.subcore_barrier()

        wid = cid * NS + sid
        ebase = wid * E_PER_W

        def chunk(kk, carry):
            off = ebase + kk * K
            pltpu.sync_copy(col_hbm.at[pl.ds(off, K)], cols_v)
            pltpu.sync_copy(row_hbm.at[pl.ds(off, K)], rows_v)
            pltpu.sync_copy(val_hbm.at[pl.ds(off, K)], vals_v)
            pltpu.async_copy(feat_hbm.at[cols_v], gath_v, sem).wait()

            def scale(g, c2):
                v16 = vals_v[pl.ds(g * L, L)]
                for j in range(L):
                    v = v16[j]
                    r = g * L + j
                    for d in range(D // L):
                        sl = pl.ds(d * L, L)
                        gath_v[r, sl] = gath_v[r, sl] * v
                return c2
            lax.fori_loop(0, K // L, scale, 0)

            pltpu.sync_copy(gath_v, acc_sh.at[rows_v], add=True)
            return carry
        lax.fori_loop(0, CHUNKS, chunk, 0)

        plsc.subcore_barrier()
        @pl.when(sid < NS - 1)
        def _():
            pltpu.sync_copy(acc_sh.at[pl.ds(base_row, ROWS_PER_TILE)],
                            out_hbm.at[cid, pl.ds(base_row, ROWS_PER_TILE)])

        @pl.when(sid == NS - 1)
        def _():
            last = N - (NS - 1) * ROWS_PER_TILE
            pltpu.sync_copy(acc_sh.at[pl.ds(base_row, last)],
                            out_hbm.at[cid, pl.ds(base_row, last)])

    return k(features, rows, cols, values)


def _tc_sum(partials):
    RB = 400

    def body(p_ref, o_ref):
        o_ref[...] = p_ref[0] + p_ref[1]

    return pl.pallas_call(
        body,
        grid=(N // RB,),
        in_specs=[pl.BlockSpec((2, RB, D), lambda i: (0, i, 0))],
        out_specs=pl.BlockSpec((RB, D), lambda i: (i, 0)),
        out_shape=jax.ShapeDtypeStruct((N, D), jnp.float32),
    )(partials)


def kernel(features, laplacianMat_indices, laplacianMat_values, selfLoop):
    del selfLoop
    rows = laplacianMat_indices[0]
    cols = laplacianMat_indices[1]
    partials = _sc_partials(features, rows, cols, laplacianMat_values)
    return _tc_sum(partials)


# speedup vs baseline: 10.9187x; 2.4402x over previous
import functools

import jax
import jax.numpy as jnp
from jax import lax
from jax.experimental import pallas as pl
from jax.experimental.pallas import tpu as pltpu
from jax.experimental.pallas import tpu_sc as plsc

N = 10000
E = 320000
D = 128
L = 16
NC = 2
NS = 16
NW = NC * NS
E_PER_W = E // NW
K = 80
CHUNKS = E_PER_W // K
RPT = 624
RPT_LAST = N - (NS - 1) * RPT


def _sc_partials(features, rows, cols, values):
    mesh = plsc.VectorSubcoreMesh(
        core_axis_name="c", subcore_axis_name="s", num_cores=NC, num_subcores=NS
    )

    @functools.partial(
        pl.kernel,
        out_type=jax.ShapeDtypeStruct((NC, N, D), jnp.float32),
        mesh=mesh,
        scratch_types=[
            pltpu.VMEM((E_PER_W,), jnp.int32),
            pltpu.VMEM((E_PER_W,), jnp.float32),
            pltpu.VMEM((K,), jnp.int32),
            pltpu.VMEM((K,), jnp.int32),
            pltpu.VMEM((K, D), jnp.float32),
            pltpu.VMEM((K, D), jnp.float32),
            pltpu.SemaphoreType.DMA,
            pltpu.SemaphoreType.DMA,
            pltpu.SemaphoreType.DMA,
            pltpu.SemaphoreType.DMA,
            pltpu.VMEM_SHARED((N, D), jnp.float32),
        ],
    )
    def k(feat_hbm, row_hbm, col_hbm, val_hbm, out_hbm,
          cols_v, vals_v, rowb0, rowb1, gath0, gath1,
          gsem0, gsem1, rsem0, rsem1, acc_sh):
        cid = lax.axis_index("c")
        sid = lax.axis_index("s")
        base_row = sid * RPT

        def zrow(i, carry):
            for d in range(D // L):
                gath0[i, pl.ds(d * L, L)] = jnp.zeros((L,), jnp.float32)
            return carry
        lax.fori_loop(0, K, zrow, 0)
        for z in range(RPT // K):
            pltpu.sync_copy(gath0, acc_sh.at[pl.ds(base_row + z * K, K)])
        nfull = (RPT // K) * K

        @pl.when(sid < NS - 1)
        def _():
            pltpu.sync_copy(gath0.at[pl.ds(0, RPT - nfull)],
                            acc_sh.at[pl.ds(base_row + nfull, RPT - nfull)])

        @pl.when(sid == NS - 1)
        def _():
            pltpu.sync_copy(gath0.at[pl.ds(0, RPT_LAST - nfull)],
                            acc_sh.at[pl.ds(base_row + nfull, RPT_LAST - nfull)])
        plsc.subcore_barrier()

        wid = cid * NS + sid
        ebase = wid * E_PER_W
        pltpu.sync_copy(col_hbm.at[pl.ds(ebase, E_PER_W)], cols_v)
        pltpu.sync_copy(val_hbm.at[pl.ds(ebase, E_PER_W)], vals_v)

        def start(kk, gath, gsem, rowb, rsem):
            pltpu.async_copy(
                feat_hbm.at[cols_v.at[pl.ds(kk * K, K)]], gath, gsem)
            pltpu.async_copy(row_hbm.at[pl.ds(ebase + kk * K, K)], rowb, rsem)

        def process(kk, gath, gsem, rowb, rsem):
            pltpu.make_async_copy(
                row_hbm.at[pl.ds(ebase + kk * K, K)], rowb, rsem).wait()
            pltpu.make_async_copy(
                feat_hbm.at[cols_v.at[pl.ds(kk * K, K)]], gath, gsem).wait()

            def scale(g, c2):
                v16 = vals_v[pl.ds(kk * K + g * L, L)]
                for j in range(L):
                    v = v16[j]
                    r = g * L + j
                    for d in range(D // L):
                        sl = pl.ds(d * L, L)
                        gath[r, sl] = gath[r, sl] * v
                return c2
            lax.fori_loop(0, K // L, scale, 0)
            pltpu.sync_copy(gath, acc_sh.at[rowb], add=True)

        start(0, gath0, gsem0, rowb0, rsem0)

        def chunk(kk, carry):
            nxt = kk + 1

            @pl.when(jnp.logical_and(nxt < CHUNKS, nxt % 2 == 0))
            def _():
                start(nxt, gath0, gsem0, rowb0, rsem0)

            @pl.when(jnp.logical_and(nxt < CHUNKS, nxt % 2 == 1))
            def _():
                start(nxt, gath1, gsem1, rowb1, rsem1)

            @pl.when(kk % 2 == 0)
            def _():
                process(kk, gath0, gsem0, rowb0, rsem0)

            @pl.when(kk % 2 == 1)
            def _():
                process(kk, gath1, gsem1, rowb1, rsem1)
            return carry
        lax.fori_loop(0, CHUNKS, chunk, 0)

        plsc.subcore_barrier()
        @pl.when(sid < NS - 1)
        def _():
            pltpu.sync_copy(acc_sh.at[pl.ds(base_row, RPT)],
                            out_hbm.at[cid, pl.ds(base_row, RPT)])

        @pl.when(sid == NS - 1)
        def _():
            pltpu.sync_copy(acc_sh.at[pl.ds(base_row, RPT_LAST)],
                            out_hbm.at[cid, pl.ds(base_row, RPT_LAST)])

    return k(features, rows, cols, values)


def _tc_sum(partials):
    RB = 400

    def body(p_ref, o_ref):
        o_ref[...] = p_ref[0] + p_ref[1]

    return pl.pallas_call(
        body,
        grid=(N // RB,),
        in_specs=[pl.BlockSpec((2, RB, D), lambda i: (0, i, 0))],
        out_specs=pl.BlockSpec((RB, D), lambda i: (i, 0)),
        out_shape=jax.ShapeDtypeStruct((N, D), jnp.float32),
    )(partials)


def kernel(features, laplacianMat_indices, laplacianMat_values, selfLoop):
    del selfLoop
    rows = laplacianMat_indices[0]
    cols = laplacianMat_indices[1]
    partials = _sc_partials(features, rows, cols, laplacianMat_values)
    return _tc_sum(partials)


# speedup vs baseline: 10.9338x; 1.0014x over previous
import functools

import jax
import jax.numpy as jnp
from jax import lax
from jax.experimental import pallas as pl
from jax.experimental.pallas import tpu as pltpu
from jax.experimental.pallas import tpu_sc as plsc

N = 10000
E = 320000
D = 128
L = 16
NC = 2
NS = 16
NW = NC * NS
E_PER_W = E // NW
K = 80
CHUNKS = E_PER_W // K
RPT = 624
RPT_LAST = N - (NS - 1) * RPT


def _sc_partials(features, rows, cols, values):
    mesh = plsc.VectorSubcoreMesh(
        core_axis_name="c", subcore_axis_name="s", num_cores=NC, num_subcores=NS
    )

    @functools.partial(
        pl.kernel,
        out_type=jax.ShapeDtypeStruct((NC, N, D), jnp.float32),
        mesh=mesh,
        scratch_types=[
            pltpu.VMEM((E_PER_W,), jnp.int32),
            pltpu.VMEM((E_PER_W,), jnp.float32),
            pltpu.VMEM((K,), jnp.int32),
            pltpu.VMEM((K,), jnp.int32),
            pltpu.VMEM((K, D), jnp.float32),
            pltpu.VMEM((K, D), jnp.float32),
            pltpu.SemaphoreType.DMA,
            pltpu.SemaphoreType.DMA,
            pltpu.SemaphoreType.DMA,
            pltpu.SemaphoreType.DMA,
            pltpu.SemaphoreType.DMA,
            pltpu.SemaphoreType.DMA,
            pltpu.VMEM_SHARED((N, D), jnp.float32),
        ],
    )
    def k(feat_hbm, row_hbm, col_hbm, val_hbm, out_hbm,
          cols_v, vals_v, rowb0, rowb1, gath0, gath1,
          gsem0, gsem1, rsem0, rsem1, ssem0, ssem1, acc_sh):
        cid = lax.axis_index("c")
        sid = lax.axis_index("s")
        base_row = sid * RPT

        def zrow(i, carry):
            for d in range(D // L):
                gath0[i, pl.ds(d * L, L)] = jnp.zeros((L,), jnp.float32)
            return carry
        lax.fori_loop(0, K, zrow, 0)
        for z in range(RPT // K):
            pltpu.sync_copy(gath0, acc_sh.at[pl.ds(base_row + z * K, K)])
        nfull = (RPT // K) * K

        @pl.when(sid < NS - 1)
        def _():
            pltpu.sync_copy(gath0.at[pl.ds(0, RPT - nfull)],
                            acc_sh.at[pl.ds(base_row + nfull, RPT - nfull)])

        @pl.when(sid == NS - 1)
        def _():
            pltpu.sync_copy(gath0.at[pl.ds(0, RPT_LAST - nfull)],
                            acc_sh.at[pl.ds(base_row + nfull, RPT_LAST - nfull)])
        plsc.subcore_barrier()

        wid = cid * NS + sid
        ebase = wid * E_PER_W
        pltpu.sync_copy(col_hbm.at[pl.ds(ebase, E_PER_W)], cols_v)
        pltpu.sync_copy(val_hbm.at[pl.ds(ebase, E_PER_W)], vals_v)

        def start(kk, gath, gsem, rowb, rsem):
            pltpu.async_copy(
                feat_hbm.at[cols_v.at[pl.ds(kk * K, K)]], gath, gsem)
            pltpu.async_copy(row_hbm.at[pl.ds(ebase + kk * K, K)], rowb, rsem)

        def drain_scatter(gath, rowb, ssem):
            pltpu.make_async_copy(gath, acc_sh.at[rowb], ssem).wait()

        def process(kk, gath, gsem, rowb, rsem, ssem):
            pltpu.make_async_copy(
                row_hbm.at[pl.ds(ebase + kk * K, K)], rowb, rsem).wait()
            pltpu.make_async_copy(
                feat_hbm.at[cols_v.at[pl.ds(kk * K, K)]], gath, gsem).wait()

            def scale(g, c2):
                v16 = vals_v[pl.ds(kk * K + g * L, L)]
                for j in range(L):
                    v = v16[j]
                    r = g * L + j
                    for d in range(D // L):
                        sl = pl.ds(d * L, L)
                        gath[r, sl] = gath[r, sl] * v
                return c2
            lax.fori_loop(0, K // L, scale, 0)
            pltpu.async_copy(gath, acc_sh.at[rowb], ssem, add=True)

        start(0, gath0, gsem0, rowb0, rsem0)

        def chunk(kk, carry):
            nxt = kk + 1

            @pl.when(jnp.logical_and(nxt < CHUNKS, nxt % 2 == 0))
            def _():
                @pl.when(nxt >= 2)
                def _():
                    drain_scatter(gath0, rowb0, ssem0)
                start(nxt, gath0, gsem0, rowb0, rsem0)

            @pl.when(jnp.logical_and(nxt < CHUNKS, nxt % 2 == 1))
            def _():
                @pl.when(nxt >= 2)
                def _():
                    drain_scatter(gath1, rowb1, ssem1)
                start(nxt, gath1, gsem1, rowb1, rsem1)

            @pl.when(kk % 2 == 0)
            def _():
                process(kk, gath0, gsem0, rowb0, rsem0, ssem0)

            @pl.when(kk % 2 == 1)
            def _():
                process(kk, gath1, gsem1, rowb1, rsem1, ssem1)
            return carry
        lax.fori_loop(0, CHUNKS, chunk, 0)
        drain_scatter(gath1, rowb1, ssem1)
        drain_scatter(gath0, rowb0, ssem0)

        plsc.subcore_barrier()
        @pl.when(sid < NS - 1)
        def _():
            pltpu.sync_copy(acc_sh.at[pl.ds(base_row, RPT)],
                            out_hbm.at[cid, pl.ds(base_row, RPT)])

        @pl.when(sid == NS - 1)
        def _():
            pltpu.sync_copy(acc_sh.at[pl.ds(base_row, RPT_LAST)],
                            out_hbm.at[cid, pl.ds(base_row, RPT_LAST)])

    return k(features, rows, cols, values)


def _tc_sum(partials):
    RB = 400

    def body(p_ref, o_ref):
        o_ref[...] = p_ref[0] + p_ref[1]

    return pl.pallas_call(
        body,
        grid=(N // RB,),
        in_specs=[pl.BlockSpec((2, RB, D), lambda i: (0, i, 0))],
        out_specs=pl.BlockSpec((RB, D), lambda i: (i, 0)),
        out_shape=jax.ShapeDtypeStruct((N, D), jnp.float32),
    )(partials)


def kernel(features, laplacianMat_indices, laplacianMat_values, selfLoop):
    del selfLoop
    rows = laplacianMat_indices[0]
    cols = laplacianMat_indices[1]
    partials = _sc_partials(features, rows, cols, laplacianMat_values)
    return _tc_sum(partials)


# speedup vs baseline: 12.2484x; 1.1202x over previous
import functools

import jax
import jax.numpy as jnp
from jax import lax
from jax.experimental import pallas as pl
from jax.experimental.pallas import tpu as pltpu
from jax.experimental.pallas import tpu_sc as plsc

N = 10000
E = 320000
D = 128
L = 16
NC = 2
NS = 16
NW = NC * NS
E_PER_W = E // NW
K = 80
CHUNKS = E_PER_W // K
RPT = 624
RPT_LAST = N - (NS - 1) * RPT
NBUF = 3


def _sc_partials(features, rows, cols, values):
    mesh = plsc.VectorSubcoreMesh(
        core_axis_name="c", subcore_axis_name="s", num_cores=NC, num_subcores=NS
    )

    @functools.partial(
        pl.kernel,
        out_type=jax.ShapeDtypeStruct((NC, N, D), jnp.float32),
        mesh=mesh,
        scratch_types=[
            pltpu.VMEM((E_PER_W,), jnp.int32),
            [pltpu.VMEM((K,), jnp.int32) for _ in range(NBUF)],
            [pltpu.VMEM((K,), jnp.float32) for _ in range(NBUF)],
            [pltpu.VMEM((K, D), jnp.float32) for _ in range(NBUF)],
            [pltpu.SemaphoreType.DMA for _ in range(NBUF)],
            [pltpu.SemaphoreType.DMA for _ in range(NBUF)],
            [pltpu.SemaphoreType.DMA for _ in range(NBUF)],
            [pltpu.SemaphoreType.DMA for _ in range(NBUF)],
            pltpu.VMEM_SHARED((N, D), jnp.float32),
        ],
    )
    def k(feat_hbm, row_hbm, col_hbm, val_hbm, out_hbm,
          cols_v, rowb, valb, gath, gsem, rsem, vsem, ssem, acc_sh):
        cid = lax.axis_index("c")
        sid = lax.axis_index("s")
        base_row = sid * RPT

        def zrow(i, carry):
            for d in range(D // L):
                gath[0][i, pl.ds(d * L, L)] = jnp.zeros((L,), jnp.float32)
            return carry
        lax.fori_loop(0, K, zrow, 0)
        for z in range(RPT // K):
            pltpu.sync_copy(gath[0], acc_sh.at[pl.ds(base_row + z * K, K)])
        nfull = (RPT // K) * K

        @pl.when(sid < NS - 1)
        def _():
            pltpu.sync_copy(gath[0].at[pl.ds(0, RPT - nfull)],
                            acc_sh.at[pl.ds(base_row + nfull, RPT - nfull)])

        @pl.when(sid == NS - 1)
        def _():
            pltpu.sync_copy(gath[0].at[pl.ds(0, RPT_LAST - nfull)],
                            acc_sh.at[pl.ds(base_row + nfull, RPT_LAST - nfull)])
        plsc.subcore_barrier()

        wid = cid * NS + sid
        ebase = wid * E_PER_W
        pltpu.sync_copy(col_hbm.at[pl.ds(ebase, E_PER_W)], cols_v)

        def start(kk, b):
            pltpu.async_copy(
                feat_hbm.at[cols_v.at[pl.ds(kk * K, K)]], gath[b], gsem[b])
            pltpu.async_copy(row_hbm.at[pl.ds(ebase + kk * K, K)],
                             rowb[b], rsem[b])
            pltpu.async_copy(val_hbm.at[pl.ds(ebase + kk * K, K)],
                             valb[b], vsem[b])

        def drain_scatter(b):
            pltpu.make_async_copy(gath[b], acc_sh.at[rowb[b]], ssem[b]).wait()

        def process(kk, b):
            pltpu.make_async_copy(row_hbm.at[pl.ds(ebase + kk * K, K)],
                                  rowb[b], rsem[b]).wait()
            pltpu.make_async_copy(val_hbm.at[pl.ds(ebase + kk * K, K)],
                                  valb[b], vsem[b]).wait()
            pltpu.make_async_copy(
                feat_hbm.at[cols_v.at[pl.ds(kk * K, K)]], gath[b],
                gsem[b]).wait()

            def scale(g, c2):
                v16 = valb[b][pl.ds(g * L, L)]
                for j in range(L):
                    v = v16[j]
                    r = g * L + j
                    for d in range(D // L):
                        sl = pl.ds(d * L, L)
                        gath[b][r, sl] = gath[b][r, sl] * v
                return c2
            lax.fori_loop(0, K // L, scale, 0)
            pltpu.async_copy(gath[b], acc_sh.at[rowb[b]], ssem[b], add=True)

        start(0, 0)
        start(1, 1)

        def chunk(kk, carry):
            for b in range(NBUF):
                @pl.when(kk % NBUF == b)
                def _(b=b):
                    process(kk, b)
                    bn = (b + 2) % NBUF

                    @pl.when(kk >= 1)
                    def _():
                        drain_scatter(bn)

                    @pl.when(kk + 2 < CHUNKS)
                    def _():
                        start(kk + 2, bn)
            return carry
        lax.fori_loop(0, CHUNKS, chunk, 0)
        drain_scatter((CHUNKS - 1) % NBUF)

        plsc.subcore_barrier()
        @pl.when(sid < NS - 1)
        def _():
            pltpu.sync_copy(acc_sh.at[pl.ds(base_row, RPT)],
                            out_hbm.at[cid, pl.ds(base_row, RPT)])

        @pl.when(sid == NS - 1)
        def _():
            pltpu.sync_copy(acc_sh.at[pl.ds(base_row, RPT_LAST)],
                            out_hbm.at[cid, pl.ds(base_row, RPT_LAST)])

    return k(features, rows, cols, values)


def _tc_sum(partials):
    RB = 400

    def body(p_ref, o_ref):
        o_ref[...] = p_ref[0] + p_ref[1]

    return pl.pallas_call(
        body,
        grid=(N // RB,),
        in_specs=[pl.BlockSpec((2, RB, D), lambda i: (0, i, 0))],
        out_specs=pl.BlockSpec((RB, D), lambda i: (i, 0)),
        out_shape=jax.ShapeDtypeStruct((N, D), jnp.float32),
    )(partials)


def kernel(features, laplacianMat_indices, laplacianMat_values, selfLoop):
    del selfLoop
    rows = laplacianMat_indices[0]
    cols = laplacianMat_indices[1]
    partials = _sc_partials(features, rows, cols, laplacianMat_values)
    return _tc_sum(partials)
